# baseline (speedup 1.0000x reference)
import functools

import jax
import jax.numpy as jnp
from jax import lax
from jax.experimental import pallas as pl
from jax.experimental.pallas import tpu as pltpu
from jax.experimental.pallas import tpu_sc as plsc

_F = 22
_NW = 32
_RC = 3200
_L = 16
_GROUPS = ((0, 2), (2, 3), (5, 3), (8, 4), (12, 4), (16, 3), (19, 3))


def _make_group_kernel(n, j0, gn):
    cpc = n // _RC
    kmax = (cpc + _NW - 1) // _NW
    assert kmax % 2 == 0
    mesh = plsc.VectorSubcoreMesh(core_axis_name="c", subcore_axis_name="s")

    @functools.partial(
        pl.kernel,
        mesh=mesh,
        out_type=[pltpu.HBM((n,), jnp.float32)] * gn,
        scratch_types=[
            pltpu.VMEM((8, _RC), jnp.float32),
            pltpu.VMEM((8, _RC), jnp.float32),
            pltpu.VMEM((8 * _RC,), jnp.float32),
            pltpu.VMEM((8 * _RC,), jnp.float32),
            pltpu.SemaphoreType.DMA,
            pltpu.SemaphoreType.DMA,
            pltpu.SemaphoreType.DMA,
            pltpu.SemaphoreType.DMA,
        ],
        compiler_params=pltpu.CompilerParams(
            needs_layout_passes=False,
            use_tc_tiling_on_sc=True,
            disable_bounds_checks=True,
        ),
    )
    def sc_group(xt_hbm, *rest):
        outs = rest[:gn]
        in_v = rest[gn:gn + 2]
        out_v = rest[gn + 2:gn + 4]
        in_sem = rest[gn + 4:gn + 6]
        out_sem = rest[gn + 6:gn + 8]
        wid = lax.axis_index("s") * 2 + lax.axis_index("c")
        nch = (cpc - wid + _NW - 1) // _NW

        def issue_in(k, b):
            i0 = (wid + k * _NW) * _RC
            pltpu.async_copy(
                xt_hbm.at[pl.ds(j0, gn), pl.ds(i0, _RC)],
                in_v[b].at[pl.ds(0, gn)],
                in_sem[b],
            )

        def out_copies(k, b):
            i0 = (wid + k * _NW) * _RC
            return [
                (out_v[b].at[pl.ds(s * _RC, _RC)],
                 outs[s].at[pl.ds(i0, _RC)])
                for s in range(gn)
            ]

        def do_slot(k, b):
            @pl.when(k >= 2)
            def _drain():
                for src, dst in out_copies(k - 2, b):
                    pltpu.make_async_copy(src, dst, out_sem[b]).wait()

            @pl.when(k < nch)
            def _work():
                pltpu.make_async_copy(
                    xt_hbm.at[pl.ds(j0, gn), pl.ds(0, _RC)],
                    in_v[b].at[pl.ds(0, gn)],
                    in_sem[b],
                ).wait()

                def depad(l, carry):
                    for s in range(gn):
                        v = in_v[b][s, pl.ds(l * _L, _L)]
                        out_v[b][pl.ds(s * _RC + l * _L, _L)] = v
                    return carry

                lax.fori_loop(0, _RC // _L, depad, 0, unroll=4)
                for src, dst in out_copies(k, b):
                    pltpu.async_copy(src, dst, out_sem[b])

            @pl.when(k + 2 < nch)
            def _next():
                issue_in(k + 2, b)

        issue_in(0, 0)
        issue_in(1, 1)

        def loop_body(i, carry):
            do_slot(i * 2, 0)
            do_slot(i * 2 + 1, 1)
            return carry

        lax.fori_loop(0, kmax // 2, loop_body, 0)

        for b in range(2):
            @pl.when(kmax - 2 + b < nch)
            def _final_drain(b=b):
                for src, dst in out_copies(kmax - 2 + b, b):
                    pltpu.make_async_copy(src, dst, out_sem[b]).wait()

    return sc_group


def kernel(inputs):
    n = inputs.shape[0]
    xt = inputs.T
    cols = []
    for j0, gn in _GROUPS:
        cols.extend(_make_group_kernel(n, j0, gn)(xt))
    return tuple(o.reshape(n, 1) for o in cols)


# speedup vs baseline: 1.0484x; 1.0484x over previous
import functools

import jax
import jax.numpy as jnp
from jax import lax
from jax.experimental import pallas as pl
from jax.experimental.pallas import tpu as pltpu
from jax.experimental.pallas import tpu_sc as plsc

_F = 22
_NW = 32
_RC = 3200
_L = 16
_GROUPS = tuple((j, 2) for j in range(0, 22, 2))


def _make_group_kernel(n, j0, gn):
    cpc = n // _RC
    kmax = (cpc + _NW - 1) // _NW
    assert kmax % 2 == 0
    mesh = plsc.VectorSubcoreMesh(core_axis_name="c", subcore_axis_name="s")

    @functools.partial(
        pl.kernel,
        mesh=mesh,
        out_type=[pltpu.HBM((n,), jnp.float32)] * gn,
        scratch_types=[
            pltpu.VMEM((8, _RC), jnp.float32),
            pltpu.VMEM((8, _RC), jnp.float32),
            pltpu.VMEM((8 * _RC,), jnp.float32),
            pltpu.VMEM((8 * _RC,), jnp.float32),
            pltpu.SemaphoreType.DMA,
            pltpu.SemaphoreType.DMA,
            pltpu.SemaphoreType.DMA,
            pltpu.SemaphoreType.DMA,
        ],
        compiler_params=pltpu.CompilerParams(
            needs_layout_passes=False,
            use_tc_tiling_on_sc=True,
            disable_bounds_checks=True,
        ),
    )
    def sc_group(xt_hbm, *rest):
        outs = rest[:gn]
        in_v = rest[gn:gn + 2]
        out_v = rest[gn + 2:gn + 4]
        in_sem = rest[gn + 4:gn + 6]
        out_sem = rest[gn + 6:gn + 8]
        wid = lax.axis_index("s") * 2 + lax.axis_index("c")
        nch = (cpc - wid + _NW - 1) // _NW

        def issue_in(k, b):
            i0 = (wid + k * _NW) * _RC
            pltpu.async_copy(
                xt_hbm.at[pl.ds(j0, gn), pl.ds(i0, _RC)],
                in_v[b].at[pl.ds(0, gn)],
                in_sem[b],
            )

        def out_copies(k, b):
            i0 = (wid + k * _NW) * _RC
            return [
                (out_v[b].at[pl.ds(s * _RC, _RC)],
                 outs[s].at[pl.ds(i0, _RC)])
                for s in range(gn)
            ]

        def do_slot(k, b):
            @pl.when(k >= 2)
            def _drain():
                for src, dst in out_copies(k - 2, b):
                    pltpu.make_async_copy(src, dst, out_sem[b]).wait()

            @pl.when(k < nch)
            def _work():
                pltpu.make_async_copy(
                    xt_hbm.at[pl.ds(j0, gn), pl.ds(0, _RC)],
                    in_v[b].at[pl.ds(0, gn)],
                    in_sem[b],
                ).wait()

                def depad(l, carry):
                    for s in range(gn):
                        v = in_v[b][s, pl.ds(l * _L, _L)]
                        out_v[b][pl.ds(s * _RC + l * _L, _L)] = v
                    return carry

                lax.fori_loop(0, _RC // _L, depad, 0, unroll=4)
                for src, dst in out_copies(k, b):
                    pltpu.async_copy(src, dst, out_sem[b])

            @pl.when(k + 2 < nch)
            def _next():
                issue_in(k + 2, b)

        issue_in(0, 0)
        issue_in(1, 1)

        def loop_body(i, carry):
            do_slot(i * 2, 0)
            do_slot(i * 2 + 1, 1)
            return carry

        lax.fori_loop(0, kmax // 2, loop_body, 0)

        for b in range(2):
            @pl.when(kmax - 2 + b < nch)
            def _final_drain(b=b):
                for src, dst in out_copies(kmax - 2 + b, b):
                    pltpu.make_async_copy(src, dst, out_sem[b]).wait()

    return sc_group


def kernel(inputs):
    n = inputs.shape[0]
    xt = inputs.T
    cols = []
    for j0, gn in _GROUPS:
        cols.extend(_make_group_kernel(n, j0, gn)(xt))
    return tuple(o.reshape(n, 1) for o in cols)


# speedup vs baseline: 1.1673x; 1.1135x over previous
import functools

import jax
import jax.numpy as jnp
from jax import lax
from jax.experimental import pallas as pl
from jax.experimental.pallas import tpu as pltpu
from jax.experimental.pallas import tpu_sc as plsc

_F = 22
_NW = 32
_RC = 3200
_L = 16
_GROUPS = tuple((j, 1) for j in range(22))


def _make_group_kernel(n, j0, gn):
    cpc = n // _RC
    kmax = (cpc + _NW - 1) // _NW
    assert kmax % 2 == 0
    mesh = plsc.VectorSubcoreMesh(core_axis_name="c", subcore_axis_name="s")

    @functools.partial(
        pl.kernel,
        mesh=mesh,
        out_type=[pltpu.HBM((n,), jnp.float32)] * gn,
        scratch_types=[
            pltpu.VMEM((8, _RC), jnp.float32),
            pltpu.VMEM((8, _RC), jnp.float32),
            pltpu.VMEM((8 * _RC,), jnp.float32),
            pltpu.VMEM((8 * _RC,), jnp.float32),
            pltpu.SemaphoreType.DMA,
            pltpu.SemaphoreType.DMA,
            pltpu.SemaphoreType.DMA,
            pltpu.SemaphoreType.DMA,
        ],
        compiler_params=pltpu.CompilerParams(
            needs_layout_passes=False,
            use_tc_tiling_on_sc=True,
            disable_bounds_checks=True,
        ),
    )
    def sc_group(xt_hbm, *rest):
        outs = rest[:gn]
        in_v = rest[gn:gn + 2]
        out_v = rest[gn + 2:gn + 4]
        in_sem = rest[gn + 4:gn + 6]
        out_sem = rest[gn + 6:gn + 8]
        wid = lax.axis_index("s") * 2 + lax.axis_index("c")
        nch = (cpc - wid + _NW - 1) // _NW

        def issue_in(k, b):
            i0 = (wid + k * _NW) * _RC
            pltpu.async_copy(
                xt_hbm.at[pl.ds(j0, gn), pl.ds(i0, _RC)],
                in_v[b].at[pl.ds(0, gn)],
                in_sem[b],
            )

        def out_copies(k, b):
            i0 = (wid + k * _NW) * _RC
            return [
                (out_v[b].at[pl.ds(s * _RC, _RC)],
                 outs[s].at[pl.ds(i0, _RC)])
                for s in range(gn)
            ]

        def do_slot(k, b):
            @pl.when(k >= 2)
            def _drain():
                for src, dst in out_copies(k - 2, b):
                    pltpu.make_async_copy(src, dst, out_sem[b]).wait()

            @pl.when(k < nch)
            def _work():
                pltpu.make_async_copy(
                    xt_hbm.at[pl.ds(j0, gn), pl.ds(0, _RC)],
                    in_v[b].at[pl.ds(0, gn)],
                    in_sem[b],
                ).wait()

                def depad(l, carry):
                    for s in range(gn):
                        v = in_v[b][s, pl.ds(l * _L, _L)]
                        out_v[b][pl.ds(s * _RC + l * _L, _L)] = v
                    return carry

                lax.fori_loop(0, _RC // _L, depad, 0, unroll=4)
                for src, dst in out_copies(k, b):
                    pltpu.async_copy(src, dst, out_sem[b])

            @pl.when(k + 2 < nch)
            def _next():
                issue_in(k + 2, b)

        issue_in(0, 0)
        issue_in(1, 1)

        def loop_body(i, carry):
            do_slot(i * 2, 0)
            do_slot(i * 2 + 1, 1)
            return carry

        lax.fori_loop(0, kmax // 2, loop_body, 0)

        for b in range(2):
            @pl.when(kmax - 2 + b < nch)
            def _final_drain(b=b):
                for src, dst in out_copies(kmax - 2 + b, b):
                    pltpu.make_async_copy(src, dst, out_sem[b]).wait()

    return sc_group


def kernel(inputs):
    n = inputs.shape[0]
    xt = inputs.T
    cols = []
    for j0, gn in _GROUPS:
        cols.extend(_make_group_kernel(n, j0, gn)(xt))
    return tuple(o.reshape(n, 1) for o in cols)


# speedup vs baseline: 1.1678x; 1.0004x over previous
import functools

import jax
import jax.numpy as jnp
from jax import lax
from jax.experimental import pallas as pl
from jax.experimental.pallas import tpu as pltpu
from jax.experimental.pallas import tpu_sc as plsc

_F = 22
_NW = 32
_RC = 3200
_L = 16
_GROUPS = tuple((j, 1) for j in range(22))


def _make_group_kernel(n, j0, gn):
    cpc = n // _RC
    kmax = (cpc + _NW - 1) // _NW
    assert kmax % 2 == 0
    mesh = plsc.VectorSubcoreMesh(core_axis_name="c", subcore_axis_name="s")

    @functools.partial(
        pl.kernel,
        mesh=mesh,
        out_type=[jax.ShapeDtypeStruct((n,), jnp.float32)] * gn,
        scratch_types=[
            pltpu.VMEM((8, _RC), jnp.float32),
            pltpu.VMEM((8, _RC), jnp.float32),
            pltpu.VMEM((8 * _RC,), jnp.float32),
            pltpu.VMEM((8 * _RC,), jnp.float32),
            pltpu.SemaphoreType.DMA,
            pltpu.SemaphoreType.DMA,
            pltpu.SemaphoreType.DMA,
            pltpu.SemaphoreType.DMA,
        ],
        compiler_params=pltpu.CompilerParams(
            needs_layout_passes=False,
            use_tc_tiling_on_sc=True,
            disable_bounds_checks=True,
        ),
    )
    def sc_group(xt_hbm, *rest):
        outs = rest[:gn]
        in_v = rest[gn:gn + 2]
        out_v = rest[gn + 2:gn + 4]
        in_sem = rest[gn + 4:gn + 6]
        out_sem = rest[gn + 6:gn + 8]
        wid = lax.axis_index("s") * 2 + lax.axis_index("c")
        nch = (cpc - wid + _NW - 1) // _NW

        def issue_in(k, b):
            i0 = (wid + k * _NW) * _RC
            pltpu.async_copy(
                xt_hbm.at[pl.ds(j0, gn), pl.ds(i0, _RC)],
                in_v[b].at[pl.ds(0, gn)],
                in_sem[b],
            )

        def out_copies(k, b):
            i0 = (wid + k * _NW) * _RC
            return [
                (out_v[b].at[pl.ds(s * _RC, _RC)],
                 outs[s].at[pl.ds(i0, _RC)])
                for s in range(gn)
            ]

        def do_slot(k, b):
            @pl.when(k >= 2)
            def _drain():
                for src, dst in out_copies(k - 2, b):
                    pltpu.make_async_copy(src, dst, out_sem[b]).wait()

            @pl.when(k < nch)
            def _work():
                pltpu.make_async_copy(
                    xt_hbm.at[pl.ds(j0, gn), pl.ds(0, _RC)],
                    in_v[b].at[pl.ds(0, gn)],
                    in_sem[b],
                ).wait()

                def depad(l, carry):
                    for s in range(gn):
                        v = in_v[b][s, pl.ds(l * _L, _L)]
                        out_v[b][pl.ds(s * _RC + l * _L, _L)] = v
                    return carry

                lax.fori_loop(0, _RC // _L, depad, 0, unroll=4)
                for src, dst in out_copies(k, b):
                    pltpu.async_copy(src, dst, out_sem[b])

            @pl.when(k + 2 < nch)
            def _next():
                issue_in(k + 2, b)

        issue_in(0, 0)
        issue_in(1, 1)

        def loop_body(i, carry):
            do_slot(i * 2, 0)
            do_slot(i * 2 + 1, 1)
            return carry

        lax.fori_loop(0, kmax // 2, loop_body, 0)

        for b in range(2):
            @pl.when(kmax - 2 + b < nch)
            def _final_drain(b=b):
                for src, dst in out_copies(kmax - 2 + b, b):
                    pltpu.make_async_copy(src, dst, out_sem[b]).wait()

    return sc_group


def kernel(inputs):
    n = inputs.shape[0]
    xt = inputs.T
    cols = []
    for j0, gn in _GROUPS:
        cols.extend(_make_group_kernel(n, j0, gn)(xt))
    return tuple(o.reshape(n, 1) for o in cols)
